# baseline (speedup 1.0000x reference)
import functools

import jax
import jax.numpy as jnp
from jax import lax
from jax.experimental import pallas as pl
from jax.experimental.pallas import tpu as pltpu
from jax.experimental.pallas import tpu_sc as plsc

_NC, _NS = 2, 16
_NW = _NC * _NS
_LANES = 16


def _sc_gather(c0, c1, pe_flat, ms0, ms1, n, h):
    b_per_w = n // _NW
    ch = 64
    nch = b_per_w // ch
    mesh = plsc.VectorSubcoreMesh(core_axis_name="c", subcore_axis_name="s")

    @functools.partial(
        pl.kernel,
        mesh=mesh,
        out_type=jax.ShapeDtypeStruct((n, h), jnp.float32),
        scratch_types=[
            pltpu.VMEM((b_per_w,), jnp.float32),
            pltpu.VMEM((b_per_w,), jnp.float32),
            pltpu.VMEM((b_per_w,), jnp.int32),
            pltpu.VMEM((ch, h), jnp.float32),
            pltpu.SemaphoreType.DMA,
        ],
    )
    def k(c0_hbm, c1_hbm, pe_hbm, enc_hbm, c0_v, c1_v, idx_v, rows_v, sem):
        wid = lax.axis_index("s") * _NC + lax.axis_index("c")
        base = wid * b_per_w
        pltpu.sync_copy(c0_hbm.at[pl.ds(base, b_per_w)], c0_v)
        pltpu.sync_copy(c1_hbm.at[pl.ds(base, b_per_w)], c1_v)

        @pl.loop(0, b_per_w, step=_LANES)
        def _(i):
            a = (c0_v[pl.ds(i, _LANES)] * float(ms0 - 1)).astype(jnp.int32)
            b = (c1_v[pl.ds(i, _LANES)] * float(ms1 - 1)).astype(jnp.int32)
            idx_v[pl.ds(i, _LANES)] = a * ms1 + b

        @pl.loop(0, nch)
        def _(c):
            off = pl.multiple_of(c * ch, 8)
            pltpu.async_copy(
                pe_hbm.at[idx_v.at[pl.ds(off, ch)]], rows_v, sem
            ).wait()
            pltpu.sync_copy(rows_v, enc_hbm.at[pl.ds(base + off, ch)])

    return k(c0, c1, pe_flat)


def _tc_combine(x2d, enc, te_row, n, d, h):
    rb = 512
    grid = (n // rb,)

    def body(x_ref, enc_ref, te_ref, o_ref):
        o_ref[:, :h] = x_ref[:, :h] + enc_ref[...]
        o_ref[:, h:] = x_ref[:, h:] + te_ref[...]

    return pl.pallas_call(
        body,
        grid=grid,
        in_specs=[
            pl.BlockSpec((rb, d), lambda i: (i, 0)),
            pl.BlockSpec((rb, h), lambda i: (i, 0)),
            pl.BlockSpec((1, h), lambda i: (0, 0)),
        ],
        out_specs=pl.BlockSpec((rb, d), lambda i: (i, 0)),
        out_shape=jax.ShapeDtypeStruct((n, d), jnp.float32),
    )(x2d, enc, te_row)


def kernel(x, spatial_coords, spatial_pe, temporal_pe, temporal_idx):
    B, S, D = x.shape
    H = D // 2
    N = B * S
    MS0, MS1 = spatial_pe.shape[0], spatial_pe.shape[1]
    MT = temporal_pe.shape[0]

    c0 = spatial_coords[..., 0].reshape(N)
    c1 = spatial_coords[..., 1].reshape(N)
    pe_flat = spatial_pe.reshape(MS0 * MS1, H)

    enc = _sc_gather(c0, c1, pe_flat, MS0, MS1, N, H)
    te_row = jax.lax.dynamic_slice_in_dim(temporal_pe, temporal_idx % MT, 1)
    out2d = _tc_combine(x.reshape(N, D), enc, te_row, N, D, H)
    return out2d.reshape(B, S, D)
